# Initial kernel scaffold; baseline (speedup 1.0000x reference)
import jax
import jax.numpy as jnp
from jax.experimental import pallas as pl


def kernel(s, r, o, t, E_re, E_im, R_re, R_im, R_no_time_re, R_no_time_im, T_re, T_im):
    raise NotImplementedError("write your pallas kernel here")


# speedup vs baseline: 8.6666x; 8.6666x over previous
import functools

import jax
import jax.numpy as jnp
from jax import lax
from jax.experimental import pallas as pl
from jax.experimental.pallas import tpu as pltpu
from jax.experimental.pallas import tpu_sc as plsc

D = 128
LANES = 16


def _make_sc_kernel(N, NC, NS, C):
    NW = NC * NS
    per_w = N // NW
    steps = per_w // C
    G = C // LANES
    mesh = plsc.VectorSubcoreMesh(core_axis_name="c", subcore_axis_name="s")

    @functools.partial(
        pl.kernel,
        out_type=jax.ShapeDtypeStruct((N,), jnp.float32),
        mesh=mesh,
        compiler_params=pltpu.CompilerParams(needs_layout_passes=False),
        scratch_types=[
            pltpu.VMEM((C,), jnp.int32),
            pltpu.VMEM((C,), jnp.int32),
            pltpu.VMEM((C,), jnp.int32),
            pltpu.VMEM((C,), jnp.int32),
            pltpu.VMEM((C, D), jnp.float32),
            pltpu.VMEM((C, D), jnp.float32),
            pltpu.VMEM((C, D), jnp.float32),
            pltpu.VMEM((C, D), jnp.float32),
            pltpu.VMEM((C, 4 * D), jnp.float32),
            pltpu.VMEM((C, 2 * D), jnp.float32),
            pltpu.VMEM((C,), jnp.float32),
            pltpu.SemaphoreType.DMA,
        ],
    )
    def sc_kernel(s_hbm, r_hbm, o_hbm, t_hbm, e_re_hbm, e_im_hbm, r4_hbm,
                  t2_hbm, out_hbm, s_v, r_v, o_v, t_v, sre_v, sim_v, ore_v,
                  oim_v, r4_v, t2_v, out_v, sem):
        wid = lax.axis_index("s") * NC + lax.axis_index("c")
        base = wid * per_w

        def step(g, carry):
            off = base + g * C
            pltpu.sync_copy(s_hbm.at[pl.ds(off, C)], s_v)
            pltpu.sync_copy(r_hbm.at[pl.ds(off, C)], r_v)
            pltpu.sync_copy(o_hbm.at[pl.ds(off, C)], o_v)
            pltpu.sync_copy(t_hbm.at[pl.ds(off, C)], t_v)
            cps = [
                pltpu.async_copy(e_re_hbm.at[s_v], sre_v, sem),
                pltpu.async_copy(e_im_hbm.at[s_v], sim_v, sem),
                pltpu.async_copy(e_re_hbm.at[o_v], ore_v, sem),
                pltpu.async_copy(e_im_hbm.at[o_v], oim_v, sem),
                pltpu.async_copy(r4_hbm.at[r_v], r4_v, sem),
                pltpu.async_copy(t2_hbm.at[t_v], t2_v, sem),
            ]
            for cp in cps:
                cp.wait()

            def group(grp, carry2):
                row0 = pl.multiple_of(grp * LANES, LANES)
                lane = lax.iota(jnp.int32, LANES)
                out_vec = jnp.zeros((LANES,), jnp.float32)
                for e in range(LANES):
                    row = row0 + e

                    def dslice(k, acc):
                        c0 = pl.multiple_of(k * LANES, LANES)
                        sre = sre_v[row, pl.ds(c0, LANES)]
                        sim = sim_v[row, pl.ds(c0, LANES)]
                        ore = ore_v[row, pl.ds(c0, LANES)]
                        oim = oim_v[row, pl.ds(c0, LANES)]
                        rre = r4_v[row, pl.ds(c0, LANES)]
                        rim = r4_v[row, pl.ds(pl.multiple_of(c0 + D, LANES), LANES)]
                        rnre = r4_v[row, pl.ds(pl.multiple_of(c0 + 2 * D, LANES), LANES)]
                        rnim = r4_v[row, pl.ds(pl.multiple_of(c0 + 3 * D, LANES), LANES)]
                        tre = t2_v[row, pl.ds(c0, LANES)]
                        tim = t2_v[row, pl.ds(pl.multiple_of(c0 + D, LANES), LANES)]
                        rrt = rre * tre - rim * tim + rnre
                        rit = rre * tim + rim * tre + rnim
                        a = sre * ore + sim * oim
                        b = sre * oim - sim * ore
                        return acc + a * rrt + b * rit

                    acc = lax.fori_loop(0, D // LANES, dslice,
                                        jnp.zeros((LANES,), jnp.float32))
                    out_vec = jnp.where(lane == e, jnp.sum(acc), out_vec)
                out_v[pl.ds(row0, LANES)] = out_vec
                return carry2

            lax.fori_loop(0, G, group, 0)
            pltpu.sync_copy(out_v, out_hbm.at[pl.ds(off, C)])
            return carry

        lax.fori_loop(0, steps, step, 0)

    return sc_kernel


def kernel(s, r, o, t, E_re, E_im, R_re, R_im, R_no_time_re, R_no_time_im,
           T_re, T_im):
    B, L = s.shape
    N = B * L
    si = s.reshape(N).astype(jnp.int32)
    ri = r.reshape(N).astype(jnp.int32)
    oi = o.reshape(N).astype(jnp.int32)
    ti = t[:, :, 0].reshape(N).astype(jnp.int32)
    r4 = jnp.concatenate([R_re, R_im, R_no_time_re, R_no_time_im], axis=1)
    t2 = jnp.concatenate([T_re, T_im], axis=1)
    info = plsc.get_sparse_core_info()
    fn = _make_sc_kernel(N, info.num_cores, info.num_subcores, 64)
    out = fn(si, ri, oi, ti, E_re, E_im, r4, t2)
    return out.reshape(B, L)


# speedup vs baseline: 14.4124x; 1.6630x over previous
import functools

import jax
import jax.numpy as jnp
from jax import lax
from jax.experimental import pallas as pl
from jax.experimental.pallas import tpu as pltpu
from jax.experimental.pallas import tpu_sc as plsc

D = 128
LANES = 16


def _make_sc_kernel(N, NC, NS, C):
    NW = NC * NS
    per_w = N // NW
    steps = per_w // C
    G = C // LANES
    assert steps % 2 == 0 and C % LANES == 0
    mesh = plsc.VectorSubcoreMesh(core_axis_name="c", subcore_axis_name="s")

    row_shapes = [(C, D), (C, D), (C, D), (C, D), (C, 4 * D), (C, 2 * D)]

    @functools.partial(
        pl.kernel,
        out_type=jax.ShapeDtypeStruct((N,), jnp.float32),
        mesh=mesh,
        compiler_params=pltpu.CompilerParams(needs_layout_passes=False),
        scratch_types=(
            [pltpu.VMEM((4 * C,), jnp.int32) for _ in range(2)]
            + [pltpu.VMEM(sh, jnp.float32) for sh in row_shapes] * 2
            + [pltpu.VMEM((C,), jnp.float32) for _ in range(2)]
            + [pltpu.SemaphoreType.DMA] * 6
        ),
    )
    def sc_kernel(idx_hbm, e_re_hbm, e_im_hbm, r4_hbm, t2_hbm, out_hbm,
                  idx0, idx1,
                  sre0, sim0, ore0, oim0, r40, t20,
                  sre1, sim1, ore1, oim1, r41, t21,
                  outv0, outv1,
                  sem_g0, sem_g1, sem_i0, sem_i1, sem_o0, sem_o1):
        wid = lax.axis_index("s") * NC + lax.axis_index("c")
        base = wid * per_w

        sets = [
            dict(idx=idx0, rows=[sre0, sim0, ore0, oim0, r40, t20],
                 outv=outv0, sem_g=sem_g0, sem_i=sem_i0, sem_o=sem_o0),
            dict(idx=idx1, rows=[sre1, sim1, ore1, oim1, r41, t21],
                 outv=outv1, sem_g=sem_g1, sem_i=sem_i1, sem_o=sem_o1),
        ]

        def fire_gathers(st):
            idx = st["idx"]
            rows = st["rows"]
            s_i = idx.at[pl.ds(0, C)]
            r_i = idx.at[pl.ds(C, C)]
            o_i = idx.at[pl.ds(2 * C, C)]
            t_i = idx.at[pl.ds(3 * C, C)]
            pltpu.async_copy(e_re_hbm.at[s_i], rows[0], st["sem_g"])
            pltpu.async_copy(e_im_hbm.at[s_i], rows[1], st["sem_g"])
            pltpu.async_copy(e_re_hbm.at[o_i], rows[2], st["sem_g"])
            pltpu.async_copy(e_im_hbm.at[o_i], rows[3], st["sem_g"])
            pltpu.async_copy(r4_hbm.at[r_i], rows[4], st["sem_g"])
            pltpu.async_copy(t2_hbm.at[t_i], rows[5], st["sem_g"])

        def drain_gathers(st):
            srcs = [e_re_hbm, e_im_hbm, e_re_hbm, e_im_hbm, r4_hbm, t2_hbm]
            for src, dst in zip(srcs, st["rows"]):
                pltpu.make_async_copy(src.at[pl.ds(0, C)], dst,
                                      st["sem_g"]).wait()

        def compute(st, off):
            rows = st["rows"]
            sre_v, sim_v, ore_v, oim_v, r4_v, t2_v = rows
            outv = st["outv"]
            lane = lax.iota(jnp.int32, LANES)

            def group(grp, carry):
                row0 = pl.multiple_of(grp * LANES, LANES)

                def elem(e, out_vec):
                    row = row0 + e
                    acc = jnp.zeros((LANES,), jnp.float32)
                    for k in range(D // LANES):
                        c0 = k * LANES
                        sre = sre_v[row, pl.ds(c0, LANES)]
                        sim = sim_v[row, pl.ds(c0, LANES)]
                        ore = ore_v[row, pl.ds(c0, LANES)]
                        oim = oim_v[row, pl.ds(c0, LANES)]
                        rre = r4_v[row, pl.ds(c0, LANES)]
                        rim = r4_v[row, pl.ds(c0 + D, LANES)]
                        rnre = r4_v[row, pl.ds(c0 + 2 * D, LANES)]
                        rnim = r4_v[row, pl.ds(c0 + 3 * D, LANES)]
                        tre = t2_v[row, pl.ds(c0, LANES)]
                        tim = t2_v[row, pl.ds(c0 + D, LANES)]
                        rrt = rre * tre - rim * tim + rnre
                        rit = rre * tim + rim * tre + rnim
                        a = sre * ore + sim * oim
                        b = sre * oim - sim * ore
                        acc = acc + a * rrt + b * rit
                    return jnp.where(lane == e, jnp.sum(acc), out_vec)

                out_vec = lax.fori_loop(0, LANES, elem,
                                        jnp.zeros((LANES,), jnp.float32))
                outv[pl.ds(row0, LANES)] = out_vec
                return carry

            lax.fori_loop(0, G, group, 0)

        def step(g, p):
            st = sets[p]
            st_n = sets[1 - p]
            off = base + g * C
            drain_gathers(st)
            q = wid * steps + g
            @pl.when(g + 2 < steps)
            def _prefetch_idx():
                pltpu.async_copy(
                    idx_hbm.at[pl.ds((q + 2) * 4 * C, 4 * C)], st["idx"],
                    st["sem_i"])
            @pl.when(g + 1 < steps)
            def _fire_next():
                pltpu.make_async_copy(
                    idx_hbm.at[pl.ds(0, 4 * C)], st_n["idx"],
                    st_n["sem_i"]).wait()
                fire_gathers(st_n)
            @pl.when(g >= 2)
            def _drain_out():
                pltpu.make_async_copy(
                    st["outv"], out_hbm.at[pl.ds(off, C)], st["sem_o"]).wait()
            compute(st, off)
            pltpu.async_copy(st["outv"], out_hbm.at[pl.ds(off, C)],
                             st["sem_o"])

        q0 = wid * steps
        pltpu.sync_copy(idx_hbm.at[pl.ds(q0 * 4 * C, 4 * C)], sets[0]["idx"])
        fire_gathers(sets[0])
        pltpu.async_copy(idx_hbm.at[pl.ds((q0 + 1) * 4 * C, 4 * C)],
                         sets[1]["idx"], sets[1]["sem_i"])

        def pair(i, carry):
            step(2 * i, 0)
            step(2 * i + 1, 1)
            return carry

        lax.fori_loop(0, steps // 2, pair, 0)

        pltpu.make_async_copy(sets[0]["outv"],
                              out_hbm.at[pl.ds(base, C)], sem_o0).wait()
        pltpu.make_async_copy(sets[1]["outv"],
                              out_hbm.at[pl.ds(base, C)], sem_o1).wait()

    return sc_kernel


def kernel(s, r, o, t, E_re, E_im, R_re, R_im, R_no_time_re, R_no_time_im,
           T_re, T_im):
    B, L = s.shape
    N = B * L
    si = s.reshape(N).astype(jnp.int32)
    ri = r.reshape(N).astype(jnp.int32)
    oi = o.reshape(N).astype(jnp.int32)
    ti = t[:, :, 0].reshape(N).astype(jnp.int32)
    C = 32
    idx4 = (jnp.stack([si, ri, oi, ti])
            .reshape(4, N // C, C).transpose(1, 0, 2).reshape(-1))
    r4 = jnp.concatenate([R_re, R_im, R_no_time_re, R_no_time_im], axis=1)
    t2 = jnp.concatenate([T_re, T_im], axis=1)
    info = plsc.get_sparse_core_info()
    fn = _make_sc_kernel(N, info.num_cores, info.num_subcores, C)
    out = fn(idx4, E_re, E_im, r4, t2)
    return out.reshape(B, L)
